# baseline (speedup 1.0000x reference)
import jax
import jax.numpy as jnp
from jax.experimental import pallas as pl

B, D_STATE, D_ACT, N_ACTIONS = 4096, 512, 16, 1000
D_OUT = D_STATE + D_ACT


def _tc_body(state_ref, w_ref, idx_ref, table_ref, rnn_ref, out_ref, rnn_out_ref):
    acc = jnp.dot(state_ref[...], w_ref[...],
                  preferred_element_type=jnp.float32)
    out_ref[:, :D_STATE] = jnp.maximum(acc, 0.0)
    idx = idx_ref[...]
    iota = jax.lax.broadcasted_iota(jnp.int32, (idx.shape[0], N_ACTIONS), 1)
    onehot = (iota == idx[:, None]).astype(jnp.float32)
    act = jnp.dot(onehot, table_ref[...], preferred_element_type=jnp.float32)
    out_ref[:, D_STATE:] = act
    rnn_out_ref[...] = rnn_ref[...]


def _tc_encode(state, w, idx, table, rnn, block_m=4096):
    grid = (B // block_m,)
    return pl.pallas_call(
        _tc_body,
        grid=grid,
        in_specs=[
            pl.BlockSpec((block_m, D_STATE), lambda i: (i, 0)),
            pl.BlockSpec((D_STATE, D_STATE), lambda i: (0, 0)),
            pl.BlockSpec((block_m,), lambda i: (i,)),
            pl.BlockSpec((N_ACTIONS, D_ACT), lambda i: (0, 0)),
            pl.BlockSpec((block_m, D_STATE), lambda i: (i, 0)),
        ],
        out_specs=[
            pl.BlockSpec((block_m, D_OUT), lambda i: (i, 0)),
            pl.BlockSpec((block_m, D_STATE), lambda i: (i, 0)),
        ],
        out_shape=[
            jax.ShapeDtypeStruct((B, D_OUT), jnp.float32),
            jax.ShapeDtypeStruct((B, D_STATE), jnp.float32),
        ],
    )(state, w, idx, table, rnn)


@jax.jit
def kernel(state, last_action, rnn_hxs, W_state, b_state, act_table):
    out, rnn_out = _tc_encode(state, W_state, last_action, act_table, rnn_hxs)
    return out, rnn_out


# speedup vs baseline: 1.1135x; 1.1135x over previous
import jax
import jax.numpy as jnp
from jax.experimental import pallas as pl

B, D_STATE, D_ACT, N_ACTIONS = 4096, 512, 16, 1000
D_OUT = D_STATE + D_ACT


def _tc_body(state_ref, w_ref, idx_ref, table_ref, rnn_ref, out_ref, rnn_out_ref):
    acc = jnp.dot(state_ref[...], w_ref[...],
                  preferred_element_type=jnp.float32)
    out_ref[:, :D_STATE] = jnp.maximum(acc, 0.0)
    idx = idx_ref[...]
    iota = jax.lax.broadcasted_iota(jnp.int32, (idx.shape[0], N_ACTIONS), 1)
    onehot = (iota == idx[:, None]).astype(jnp.float32)
    act = jnp.dot(onehot, table_ref[...], preferred_element_type=jnp.float32)
    out_ref[:, D_STATE:] = act
    rnn_out_ref[...] = rnn_ref[...]


def _tc_encode(state, w, idx, table, rnn, block_m=2048):
    grid = (B // block_m,)
    return pl.pallas_call(
        _tc_body,
        grid=grid,
        in_specs=[
            pl.BlockSpec((block_m, D_STATE), lambda i: (i, 0)),
            pl.BlockSpec((D_STATE, D_STATE), lambda i: (0, 0)),
            pl.BlockSpec((block_m,), lambda i: (i,)),
            pl.BlockSpec((N_ACTIONS, D_ACT), lambda i: (0, 0)),
            pl.BlockSpec((block_m, D_STATE), lambda i: (i, 0)),
        ],
        out_specs=[
            pl.BlockSpec((block_m, D_OUT), lambda i: (i, 0)),
            pl.BlockSpec((block_m, D_STATE), lambda i: (i, 0)),
        ],
        out_shape=[
            jax.ShapeDtypeStruct((B, D_OUT), jnp.float32),
            jax.ShapeDtypeStruct((B, D_STATE), jnp.float32),
        ],
    )(state, w, idx, table, rnn)


@jax.jit
def kernel(state, last_action, rnn_hxs, W_state, b_state, act_table):
    out, rnn_out = _tc_encode(state, W_state, last_action, act_table, rnn_hxs)
    return out, rnn_out
